# baseline (speedup 1.0000x reference)
import jax
import jax.numpy as jnp
from jax import lax
from jax.experimental import pallas as pl
from jax.experimental.pallas import tpu as pltpu
from jax.experimental.pallas import tpu_sc as plsc

N = 50000
D_FEAT = 128
DEG = 32
D_EDGE = 16
OUT_HALF = 64
MID = 96
BLK = 2048

_LANES = 128
_NPAD = ((N + _LANES - 1) // _LANES) * _LANES
_CH = _NPAD // _LANES
_NW = 32
_TRIPS = (_CH + _NW - 1) // _NW
_HROWS = DEG * D_EDGE // 2


def _make_sc_agg(ch0, nch):
    trips = (nch + _NW - 1) // _NW

    def _sc_agg(x_hbm, agg_hbm, buf0, buf1, ob0, ob1,
                isem0, isem1, osem0, osem1):
        w = lax.axis_index("s") * 2 + lax.axis_index("c")
        bufs = (buf0, buf1)
        obs = (ob0, ob1)
        isems = (isem0, isem1)
        osems = (osem0, osem1)

        def chunk(t):
            return w + _NW * t

        def start_in(t, h, b):
            pltpu.async_copy(
                x_hbm.at[pl.ds(h * _HROWS, _HROWS),
                         pl.ds((ch0 + chunk(t)) * _LANES, _LANES)],
                bufs[b], isems[b])

        @pl.when(chunk(0) < nch)
        def _():
            start_in(0, 0, 0)

        for t in range(trips):
            ob = obs[t % 2]
            for h in range(2):
                b = (2 * t + h) % 2
                nt, nh = (t, 1) if h == 0 else (t + 1, 0)
                if nt < trips:
                    @pl.when(chunk(nt) < nch)
                    def _(nt=nt, nh=nh, nb=1 - b):
                        start_in(nt, nh, nb)

                @pl.when(chunk(t) < nch)
                def _(t=t, h=h, b=b, ob=ob):
                    pltpu.make_async_copy(
                        x_hbm.at[pl.ds(0, _HROWS), pl.ds(0, _LANES)],
                        bufs[b], isems[b]).wait()
                    if h == 0 and t >= 2:
                        pltpu.make_async_copy(
                            ob, agg_hbm.at[:, pl.ds(0, _LANES)],
                            osems[t % 2]).wait()

                    def gbody(g, carry):
                        off = g * D_EDGE

                        def fbody(f4, carry2):
                            for df in range(4):
                                f = f4 * 4 + df
                                acc4 = []
                                for k in range(4):
                                    a = bufs[b][k * D_EDGE + f, pl.ds(off, D_EDGE)]
                                    for m in range(1, 4):
                                        a = a + bufs[b][(m * 4 + k) * D_EDGE + f,
                                                        pl.ds(off, D_EDGE)]
                                    acc4.append(a)
                                s = (acc4[0] + acc4[1]) + (acc4[2] + acc4[3])
                                if h == 0:
                                    ob[f, pl.ds(off, D_EDGE)] = s
                                else:
                                    ob[f, pl.ds(off, D_EDGE)] = (
                                        ob[f, pl.ds(off, D_EDGE)] + s)
                            return carry2

                        lax.fori_loop(0, D_EDGE // 4, fbody, 0)
                        return carry

                    lax.fori_loop(0, _LANES // D_EDGE, gbody, 0)
                    if h == 1:
                        pltpu.async_copy(
                            ob, agg_hbm.at[:, pl.ds(chunk(t) * _LANES, _LANES)],
                            osems[t % 2])

        for b in range(2):
            pltpu.make_async_copy(
                obs[b], agg_hbm.at[:, pl.ds(0, _LANES)], osems[b]).wait()

    return _sc_agg


def _sc_aggregate(x, ch0, nch):
    mesh = plsc.VectorSubcoreMesh(core_axis_name="c", subcore_axis_name="s")
    return pl.kernel(
        _make_sc_agg(ch0, nch),
        out_type=jax.ShapeDtypeStruct((D_EDGE, nch * _LANES), jnp.float32),
        mesh=mesh,
        scratch_types=[
            pltpu.VMEM((_HROWS, _LANES), jnp.float32),
            pltpu.VMEM((_HROWS, _LANES), jnp.float32),
            pltpu.VMEM((D_EDGE, _LANES), jnp.float32),
            pltpu.VMEM((D_EDGE, _LANES), jnp.float32),
            pltpu.SemaphoreType.DMA,
            pltpu.SemaphoreType.DMA,
            pltpu.SemaphoreType.DMA,
            pltpu.SemaphoreType.DMA,
        ],
    )(x)


TC1_BLK = 4096
BLK_A = 16
CH_A = BLK_A * BLK // _LANES
CH_B = _CH - CH_A


def _full(shape):
    return pl.BlockSpec(shape, lambda i: (0,) * len(shape))


def _tc1_body(nf, w1at, b1a, w1bt, b1b, r1out):
    h = jnp.maximum(
        jnp.dot(nf[...], w1at[...], preferred_element_type=jnp.float32) + b1a[...], 0.0)
    r1out[...] = jnp.tanh(
        jnp.dot(h, w1bt[...], preferred_element_type=jnp.float32) + b1b[...])


def _tc2_body(r1, aggt, w2at, b2a, w2bt, b2b, out):
    agg = aggt[...].T
    h2 = jnp.maximum(
        jnp.dot(agg, w2at[...], preferred_element_type=jnp.float32) + b2a[...], 0.0)
    r2 = jnp.tanh(jnp.dot(h2, w2bt[...], preferred_element_type=jnp.float32) + b2b[...])
    res = jnp.concatenate([r1[...], r2], axis=1)
    inv = jax.lax.rsqrt(jnp.sum(res * res, axis=1, keepdims=True))
    out[...] = res * inv


def _tc2b_body(prev, r1, aggt, w2at, b2a, w2bt, b2b, out):
    del prev
    _tc2_body(r1, aggt, w2at, b2a, w2bt, b2b, out)


_TC_PARAMS = pltpu.CompilerParams(dimension_semantics=("parallel",))
_TC2_PARAMS = pltpu.CompilerParams(
    dimension_semantics=("parallel",), fuse_transposed_lhs_in_matmul=True)


def kernel(node_features, mailbox, W1a, b1a, W1b, b1b, W2a, b2a, W2b, b2b):
    n = node_features.shape[0]

    x = mailbox.transpose(1, 2, 0).reshape(DEG * D_EDGE, n)
    aggt_a = _sc_aggregate(x, 0, CH_A)
    aggt_b = _sc_aggregate(x, CH_A, CH_B)

    w2at = W2a.T
    b2a2 = b2a.reshape(1, OUT_HALF)
    w2bt = W2b.T
    b2b2 = b2b.reshape(1, OUT_HALF)

    r1 = pl.pallas_call(
        _tc1_body,
        grid=(pl.cdiv(n, TC1_BLK),),
        in_specs=[
            pl.BlockSpec((TC1_BLK, D_FEAT), lambda i: (i, 0)),
            _full((D_FEAT, MID)),
            _full((1, MID)),
            _full((MID, OUT_HALF)),
            _full((1, OUT_HALF)),
        ],
        out_specs=pl.BlockSpec((TC1_BLK, OUT_HALF), lambda i: (i, 0)),
        out_shape=jax.ShapeDtypeStruct((n, OUT_HALF), jnp.float32),
        compiler_params=_TC_PARAMS,
    )(node_features,
      W1a.T, b1a.reshape(1, MID),
      W1b.T, b1b.reshape(1, OUT_HALF))

    out_a = pl.pallas_call(
        _tc2_body,
        grid=(BLK_A,),
        in_specs=[
            pl.BlockSpec((BLK, OUT_HALF), lambda i: (i, 0)),
            pl.BlockSpec((D_EDGE, BLK), lambda i: (0, i)),
            _full((D_EDGE, OUT_HALF)),
            _full((1, OUT_HALF)),
            _full((OUT_HALF, OUT_HALF)),
            _full((1, OUT_HALF)),
        ],
        out_specs=pl.BlockSpec((BLK, D_FEAT), lambda i: (i, 0)),
        out_shape=jax.ShapeDtypeStruct((n, D_FEAT), jnp.float32),
        compiler_params=_TC2_PARAMS,
    )(r1, aggt_a, w2at, b2a2, w2bt, b2b2)

    nb_blocks = pl.cdiv(n - BLK_A * BLK, BLK)
    out = pl.pallas_call(
        _tc2b_body,
        grid=(nb_blocks,),
        in_specs=[
            pl.BlockSpec((8, D_FEAT), lambda i: (0, 0)),
            pl.BlockSpec((BLK, OUT_HALF), lambda i: (i + BLK_A, 0)),
            pl.BlockSpec((D_EDGE, BLK), lambda i: (0, i)),
            _full((D_EDGE, OUT_HALF)),
            _full((1, OUT_HALF)),
            _full((OUT_HALF, OUT_HALF)),
            _full((1, OUT_HALF)),
        ],
        out_specs=pl.BlockSpec((BLK, D_FEAT), lambda i: (i + BLK_A, 0)),
        out_shape=jax.ShapeDtypeStruct((n, D_FEAT), jnp.float32),
        input_output_aliases={0: 0},
        compiler_params=_TC2_PARAMS,
    )(out_a, r1, aggt_b, w2at, b2a2, w2bt, b2b2)
    return out


# speedup vs baseline: 1.1710x; 1.1710x over previous
import jax
import jax.numpy as jnp
from jax import lax
from jax.experimental import pallas as pl
from jax.experimental.pallas import tpu as pltpu
from jax.experimental.pallas import tpu_sc as plsc

N = 50000
D_FEAT = 128
DEG = 32
D_EDGE = 16
OUT_HALF = 64
MID = 96
BLK = 2048

_LANES = 128
_NPAD = ((N + _LANES - 1) // _LANES) * _LANES
_CH = _NPAD // _LANES
_NW = 32
_TRIPS = (_CH + _NW - 1) // _NW
_HROWS = DEG * D_EDGE // 2


def _make_sc_agg(ch0, nch):
    trips = (nch + _NW - 1) // _NW

    def _sc_agg(x_hbm, agg_hbm, buf0, buf1, ob0, ob1,
                isem0, isem1, osem0, osem1):
        w = lax.axis_index("s") * 2 + lax.axis_index("c")
        bufs = (buf0, buf1)
        obs = (ob0, ob1)
        isems = (isem0, isem1)
        osems = (osem0, osem1)

        def chunk(t):
            return w + _NW * t

        def start_in(t, h, b):
            pltpu.async_copy(
                x_hbm.at[pl.ds(h * _HROWS, _HROWS),
                         pl.ds((ch0 + chunk(t)) * _LANES, _LANES)],
                bufs[b], isems[b])

        @pl.when(chunk(0) < nch)
        def _():
            start_in(0, 0, 0)

        for t in range(trips):
            ob = obs[t % 2]
            for h in range(2):
                b = (2 * t + h) % 2
                nt, nh = (t, 1) if h == 0 else (t + 1, 0)
                if nt < trips:
                    @pl.when(chunk(nt) < nch)
                    def _(nt=nt, nh=nh, nb=1 - b):
                        start_in(nt, nh, nb)

                @pl.when(chunk(t) < nch)
                def _(t=t, h=h, b=b, ob=ob):
                    pltpu.make_async_copy(
                        x_hbm.at[pl.ds(0, _HROWS), pl.ds(0, _LANES)],
                        bufs[b], isems[b]).wait()
                    if h == 0 and t >= 2:
                        pltpu.make_async_copy(
                            ob, agg_hbm.at[:, pl.ds(0, _LANES)],
                            osems[t % 2]).wait()

                    def gbody(g, carry):
                        off = g * D_EDGE

                        def fbody(f4, carry2):
                            for df in range(4):
                                f = f4 * 4 + df
                                acc4 = []
                                for k in range(4):
                                    a = bufs[b][k * D_EDGE + f, pl.ds(off, D_EDGE)]
                                    for m in range(1, 4):
                                        a = a + bufs[b][(m * 4 + k) * D_EDGE + f,
                                                        pl.ds(off, D_EDGE)]
                                    acc4.append(a)
                                s = (acc4[0] + acc4[1]) + (acc4[2] + acc4[3])
                                if h == 0:
                                    ob[f, pl.ds(off, D_EDGE)] = s
                                else:
                                    ob[f, pl.ds(off, D_EDGE)] = (
                                        ob[f, pl.ds(off, D_EDGE)] + s)
                            return carry2

                        lax.fori_loop(0, D_EDGE // 4, fbody, 0)
                        return carry

                    lax.fori_loop(0, _LANES // D_EDGE, gbody, 0)
                    if h == 1:
                        pltpu.async_copy(
                            ob, agg_hbm.at[:, pl.ds(chunk(t) * _LANES, _LANES)],
                            osems[t % 2])

        for b in range(2):
            pltpu.make_async_copy(
                obs[b], agg_hbm.at[:, pl.ds(0, _LANES)], osems[b]).wait()

    return _sc_agg


def _sc_aggregate(x, ch0, nch):
    mesh = plsc.VectorSubcoreMesh(core_axis_name="c", subcore_axis_name="s")
    return pl.kernel(
        _make_sc_agg(ch0, nch),
        out_type=jax.ShapeDtypeStruct((D_EDGE, nch * _LANES), jnp.float32),
        mesh=mesh,
        scratch_types=[
            pltpu.VMEM((_HROWS, _LANES), jnp.float32),
            pltpu.VMEM((_HROWS, _LANES), jnp.float32),
            pltpu.VMEM((D_EDGE, _LANES), jnp.float32),
            pltpu.VMEM((D_EDGE, _LANES), jnp.float32),
            pltpu.SemaphoreType.DMA,
            pltpu.SemaphoreType.DMA,
            pltpu.SemaphoreType.DMA,
            pltpu.SemaphoreType.DMA,
        ],
    )(x)


TC1_BLK = 4096
BLK_A = 13
BLK_B = 5
CH_A = BLK_A * BLK // _LANES
CH_B = BLK_B * BLK // _LANES
_OFF_C = BLK_A + BLK_B


def _full(shape):
    return pl.BlockSpec(shape, lambda i: (0,) * len(shape))


def _tc1_body(nf, w1at, b1a, w1bt, b1b, r1out):
    h = jnp.maximum(
        jnp.dot(nf[...], w1at[...], preferred_element_type=jnp.float32) + b1a[...], 0.0)
    r1out[...] = jnp.tanh(
        jnp.dot(h, w1bt[...], preferred_element_type=jnp.float32) + b1b[...])


def _finish(r1, aggt, w2at, b2a, w2bt, b2b, out):
    agg = aggt.T
    h2 = jnp.maximum(
        jnp.dot(agg, w2at[...], preferred_element_type=jnp.float32) + b2a[...], 0.0)
    r2 = jnp.tanh(jnp.dot(h2, w2bt[...], preferred_element_type=jnp.float32) + b2b[...])
    res = jnp.concatenate([r1[...], r2], axis=1)
    inv = jax.lax.rsqrt(jnp.sum(res * res, axis=1, keepdims=True))
    out[...] = res * inv


def _tc2c_body(prev, r1, xs, w2at, b2a, w2bt, b2b, out):
    del prev
    mb = xs[...]
    parts = [mb[j * D_EDGE:(j + 1) * D_EDGE, :] for j in range(DEG)]
    while len(parts) > 1:
        parts = [parts[i] + parts[i + 1] for i in range(0, len(parts), 2)]
    _finish(r1, parts[0], w2at, b2a, w2bt, b2b, out)


def _tc2_body(prev, r1, aggt, w2at, b2a, w2bt, b2b, out):
    del prev
    _finish(r1, aggt[...], w2at, b2a, w2bt, b2b, out)


_TC_PARAMS = pltpu.CompilerParams(dimension_semantics=("parallel",))
_TC2_PARAMS = pltpu.CompilerParams(
    dimension_semantics=("parallel",), fuse_transposed_lhs_in_matmul=True)


def kernel(node_features, mailbox, W1a, b1a, W1b, b1b, W2a, b2a, W2b, b2b):
    n = node_features.shape[0]

    x = mailbox.transpose(1, 2, 0).reshape(DEG * D_EDGE, n)
    aggt_a = _sc_aggregate(x, 0, CH_A)
    aggt_b = _sc_aggregate(x, CH_A, CH_B)

    w2at = W2a.T
    b2a2 = b2a.reshape(1, OUT_HALF)
    w2bt = W2b.T
    b2b2 = b2b.reshape(1, OUT_HALF)

    r1 = pl.pallas_call(
        _tc1_body,
        grid=(pl.cdiv(n, TC1_BLK),),
        in_specs=[
            pl.BlockSpec((TC1_BLK, D_FEAT), lambda i: (i, 0)),
            _full((D_FEAT, MID)),
            _full((1, MID)),
            _full((MID, OUT_HALF)),
            _full((1, OUT_HALF)),
        ],
        out_specs=pl.BlockSpec((TC1_BLK, OUT_HALF), lambda i: (i, 0)),
        out_shape=jax.ShapeDtypeStruct((n, OUT_HALF), jnp.float32),
        compiler_params=_TC_PARAMS,
    )(node_features,
      W1a.T, b1a.reshape(1, MID),
      W1b.T, b1b.reshape(1, OUT_HALF))

    seed = jnp.zeros((8, D_FEAT), jnp.float32)
    nc_blocks = pl.cdiv(n - _OFF_C * BLK, BLK)
    out_c = pl.pallas_call(
        _tc2c_body,
        grid=(nc_blocks,),
        in_specs=[
            pl.BlockSpec((8, D_FEAT), lambda i: (0, 0)),
            pl.BlockSpec((BLK, OUT_HALF), lambda i: (i + _OFF_C, 0)),
            pl.BlockSpec((DEG * D_EDGE, BLK), lambda i: (0, i + _OFF_C)),
            _full((D_EDGE, OUT_HALF)),
            _full((1, OUT_HALF)),
            _full((OUT_HALF, OUT_HALF)),
            _full((1, OUT_HALF)),
        ],
        out_specs=pl.BlockSpec((BLK, D_FEAT), lambda i: (i + _OFF_C, 0)),
        out_shape=jax.ShapeDtypeStruct((n, D_FEAT), jnp.float32),
        input_output_aliases={},
        compiler_params=_TC2_PARAMS,
    )(seed, r1, x, w2at, b2a2, w2bt, b2b2)

    out_a = pl.pallas_call(
        _tc2_body,
        grid=(BLK_A,),
        in_specs=[
            pl.BlockSpec((8, D_FEAT), lambda i: (0, 0)),
            pl.BlockSpec((BLK, OUT_HALF), lambda i: (i, 0)),
            pl.BlockSpec((D_EDGE, BLK), lambda i: (0, i)),
            _full((D_EDGE, OUT_HALF)),
            _full((1, OUT_HALF)),
            _full((OUT_HALF, OUT_HALF)),
            _full((1, OUT_HALF)),
        ],
        out_specs=pl.BlockSpec((BLK, D_FEAT), lambda i: (i, 0)),
        out_shape=jax.ShapeDtypeStruct((n, D_FEAT), jnp.float32),
        input_output_aliases={0: 0},
        compiler_params=_TC2_PARAMS,
    )(out_c, r1, aggt_a, w2at, b2a2, w2bt, b2b2)

    out = pl.pallas_call(
        _tc2_body,
        grid=(BLK_B,),
        in_specs=[
            pl.BlockSpec((8, D_FEAT), lambda i: (0, 0)),
            pl.BlockSpec((BLK, OUT_HALF), lambda i: (i + BLK_A, 0)),
            pl.BlockSpec((D_EDGE, BLK), lambda i: (0, i)),
            _full((D_EDGE, OUT_HALF)),
            _full((1, OUT_HALF)),
            _full((OUT_HALF, OUT_HALF)),
            _full((1, OUT_HALF)),
        ],
        out_specs=pl.BlockSpec((BLK, D_FEAT), lambda i: (i + BLK_A, 0)),
        out_shape=jax.ShapeDtypeStruct((n, D_FEAT), jnp.float32),
        input_output_aliases={0: 0},
        compiler_params=_TC2_PARAMS,
    )(out_a, r1, aggt_b, w2at, b2a2, w2bt, b2b2)
    return out


# speedup vs baseline: 1.2141x; 1.0368x over previous
import jax
import jax.numpy as jnp
from jax import lax
from jax.experimental import pallas as pl
from jax.experimental.pallas import tpu as pltpu
from jax.experimental.pallas import tpu_sc as plsc

N = 50000
D_FEAT = 128
DEG = 32
D_EDGE = 16
OUT_HALF = 64
MID = 96
BLK = 2048

_LANES = 128
_NPAD = ((N + _LANES - 1) // _LANES) * _LANES
_CH = _NPAD // _LANES
_NW = 32
_TRIPS = (_CH + _NW - 1) // _NW
_HROWS = DEG * D_EDGE // 2


def _make_sc_agg(ch0, nch):
    trips = (nch + _NW - 1) // _NW

    def _sc_agg(x_hbm, agg_hbm, buf0, buf1, ob0, ob1,
                isem0, isem1, osem0, osem1):
        w = lax.axis_index("s") * 2 + lax.axis_index("c")
        bufs = (buf0, buf1)
        obs = (ob0, ob1)
        isems = (isem0, isem1)
        osems = (osem0, osem1)

        def chunk(t):
            return w + _NW * t

        def start_in(t, h, b):
            pltpu.async_copy(
                x_hbm.at[pl.ds(h * _HROWS, _HROWS),
                         pl.ds((ch0 + chunk(t)) * _LANES, _LANES)],
                bufs[b], isems[b])

        @pl.when(chunk(0) < nch)
        def _():
            start_in(0, 0, 0)

        for t in range(trips):
            ob = obs[t % 2]
            for h in range(2):
                b = (2 * t + h) % 2
                nt, nh = (t, 1) if h == 0 else (t + 1, 0)
                if nt < trips:
                    @pl.when(chunk(nt) < nch)
                    def _(nt=nt, nh=nh, nb=1 - b):
                        start_in(nt, nh, nb)

                @pl.when(chunk(t) < nch)
                def _(t=t, h=h, b=b, ob=ob):
                    pltpu.make_async_copy(
                        x_hbm.at[pl.ds(0, _HROWS), pl.ds(0, _LANES)],
                        bufs[b], isems[b]).wait()
                    if h == 0 and t >= 2:
                        pltpu.make_async_copy(
                            ob, agg_hbm.at[:, pl.ds(0, _LANES)],
                            osems[t % 2]).wait()

                    def gbody(g, carry):
                        off = g * D_EDGE

                        def fbody(f4, carry2):
                            for df in range(4):
                                f = f4 * 4 + df
                                acc4 = []
                                for k in range(4):
                                    a = bufs[b][k * D_EDGE + f, pl.ds(off, D_EDGE)]
                                    for m in range(1, 4):
                                        a = a + bufs[b][(m * 4 + k) * D_EDGE + f,
                                                        pl.ds(off, D_EDGE)]
                                    acc4.append(a)
                                s = (acc4[0] + acc4[1]) + (acc4[2] + acc4[3])
                                if h == 0:
                                    ob[f, pl.ds(off, D_EDGE)] = s
                                else:
                                    ob[f, pl.ds(off, D_EDGE)] = (
                                        ob[f, pl.ds(off, D_EDGE)] + s)
                            return carry2

                        lax.fori_loop(0, D_EDGE // 4, fbody, 0)
                        return carry

                    lax.fori_loop(0, _LANES // D_EDGE, gbody, 0)
                    if h == 1:
                        pltpu.async_copy(
                            ob, agg_hbm.at[:, pl.ds(chunk(t) * _LANES, _LANES)],
                            osems[t % 2])

        for b in range(2):
            pltpu.make_async_copy(
                obs[b], agg_hbm.at[:, pl.ds(0, _LANES)], osems[b]).wait()

    return _sc_agg


def _sc_aggregate(x, ch0, nch):
    mesh = plsc.VectorSubcoreMesh(core_axis_name="c", subcore_axis_name="s")
    return pl.kernel(
        _make_sc_agg(ch0, nch),
        out_type=jax.ShapeDtypeStruct((D_EDGE, nch * _LANES), jnp.float32),
        mesh=mesh,
        scratch_types=[
            pltpu.VMEM((_HROWS, _LANES), jnp.float32),
            pltpu.VMEM((_HROWS, _LANES), jnp.float32),
            pltpu.VMEM((D_EDGE, _LANES), jnp.float32),
            pltpu.VMEM((D_EDGE, _LANES), jnp.float32),
            pltpu.SemaphoreType.DMA,
            pltpu.SemaphoreType.DMA,
            pltpu.SemaphoreType.DMA,
            pltpu.SemaphoreType.DMA,
        ],
    )(x)


TC1_BLK = 4096
BLK_A = 13
BLK_B = 5
CH_A = BLK_A * BLK // _LANES
CH_B = BLK_B * BLK // _LANES
_OFF_C = BLK_A + BLK_B


def _full(shape):
    return pl.BlockSpec(shape, lambda i: (0,) * len(shape))


def _tc1_body(nf, w1at, b1a, w1bt, b1b, r1out):
    h = jnp.maximum(
        jnp.dot(nf[...], w1at[...], preferred_element_type=jnp.float32) + b1a[...], 0.0)
    r1out[...] = jnp.tanh(
        jnp.dot(h, w1bt[...], preferred_element_type=jnp.float32)
        + b1b[...]).astype(jnp.bfloat16)


def _finish(r1, aggt, w2at, b2a, w2bt, b2b, out):
    agg = aggt.T
    h2 = jnp.maximum(
        jnp.dot(agg, w2at[...], preferred_element_type=jnp.float32) + b2a[...], 0.0)
    r2 = jnp.tanh(jnp.dot(h2, w2bt[...], preferred_element_type=jnp.float32) + b2b[...])
    res = jnp.concatenate([r1[...].astype(jnp.float32), r2], axis=1)
    inv = jax.lax.rsqrt(jnp.sum(res * res, axis=1, keepdims=True))
    out[...] = res * inv


def _tc2c_body(prev, r1, xs, w2at, b2a, w2bt, b2b, out):
    del prev
    mb = xs[...]
    parts = [mb[j * D_EDGE:(j + 1) * D_EDGE, :] for j in range(DEG)]
    while len(parts) > 1:
        parts = [parts[i] + parts[i + 1] for i in range(0, len(parts), 2)]
    _finish(r1, parts[0], w2at, b2a, w2bt, b2b, out)


def _tc2_body(prev, r1, aggt, w2at, b2a, w2bt, b2b, out):
    del prev
    _finish(r1, aggt[...], w2at, b2a, w2bt, b2b, out)


_TC_PARAMS = pltpu.CompilerParams(dimension_semantics=("parallel",))
_TC2_PARAMS = pltpu.CompilerParams(
    dimension_semantics=("parallel",), fuse_transposed_lhs_in_matmul=True)


def kernel(node_features, mailbox, W1a, b1a, W1b, b1b, W2a, b2a, W2b, b2b):
    n = node_features.shape[0]

    x = mailbox.transpose(1, 2, 0).reshape(DEG * D_EDGE, n)
    aggt_a = _sc_aggregate(x, 0, CH_A)
    aggt_b = _sc_aggregate(x, CH_A, CH_B)

    w2at = W2a.T
    b2a2 = b2a.reshape(1, OUT_HALF)
    w2bt = W2b.T
    b2b2 = b2b.reshape(1, OUT_HALF)

    r1 = pl.pallas_call(
        _tc1_body,
        grid=(pl.cdiv(n, TC1_BLK),),
        in_specs=[
            pl.BlockSpec((TC1_BLK, D_FEAT), lambda i: (i, 0)),
            _full((D_FEAT, MID)),
            _full((1, MID)),
            _full((MID, OUT_HALF)),
            _full((1, OUT_HALF)),
        ],
        out_specs=pl.BlockSpec((TC1_BLK, OUT_HALF), lambda i: (i, 0)),
        out_shape=jax.ShapeDtypeStruct((n, OUT_HALF), jnp.bfloat16),
        compiler_params=_TC_PARAMS,
    )(node_features,
      W1a.T, b1a.reshape(1, MID),
      W1b.T, b1b.reshape(1, OUT_HALF))

    seed = jnp.zeros((8, D_FEAT), jnp.float32)
    nc_blocks = pl.cdiv(n - _OFF_C * BLK, BLK)
    out_c = pl.pallas_call(
        _tc2c_body,
        grid=(nc_blocks,),
        in_specs=[
            pl.BlockSpec((8, D_FEAT), lambda i: (0, 0)),
            pl.BlockSpec((BLK, OUT_HALF), lambda i: (i + _OFF_C, 0)),
            pl.BlockSpec((DEG * D_EDGE, BLK), lambda i: (0, i + _OFF_C)),
            _full((D_EDGE, OUT_HALF)),
            _full((1, OUT_HALF)),
            _full((OUT_HALF, OUT_HALF)),
            _full((1, OUT_HALF)),
        ],
        out_specs=pl.BlockSpec((BLK, D_FEAT), lambda i: (i + _OFF_C, 0)),
        out_shape=jax.ShapeDtypeStruct((n, D_FEAT), jnp.float32),
        input_output_aliases={},
        compiler_params=_TC2_PARAMS,
    )(seed, r1, x, w2at, b2a2, w2bt, b2b2)

    out_a = pl.pallas_call(
        _tc2_body,
        grid=(BLK_A,),
        in_specs=[
            pl.BlockSpec((8, D_FEAT), lambda i: (0, 0)),
            pl.BlockSpec((BLK, OUT_HALF), lambda i: (i, 0)),
            pl.BlockSpec((D_EDGE, BLK), lambda i: (0, i)),
            _full((D_EDGE, OUT_HALF)),
            _full((1, OUT_HALF)),
            _full((OUT_HALF, OUT_HALF)),
            _full((1, OUT_HALF)),
        ],
        out_specs=pl.BlockSpec((BLK, D_FEAT), lambda i: (i, 0)),
        out_shape=jax.ShapeDtypeStruct((n, D_FEAT), jnp.float32),
        input_output_aliases={0: 0},
        compiler_params=_TC2_PARAMS,
    )(out_c, r1, aggt_a, w2at, b2a2, w2bt, b2b2)

    out = pl.pallas_call(
        _tc2_body,
        grid=(BLK_B,),
        in_specs=[
            pl.BlockSpec((8, D_FEAT), lambda i: (0, 0)),
            pl.BlockSpec((BLK, OUT_HALF), lambda i: (i + BLK_A, 0)),
            pl.BlockSpec((D_EDGE, BLK), lambda i: (0, i)),
            _full((D_EDGE, OUT_HALF)),
            _full((1, OUT_HALF)),
            _full((OUT_HALF, OUT_HALF)),
            _full((1, OUT_HALF)),
        ],
        out_specs=pl.BlockSpec((BLK, D_FEAT), lambda i: (i + BLK_A, 0)),
        out_shape=jax.ShapeDtypeStruct((n, D_FEAT), jnp.float32),
        input_output_aliases={0: 0},
        compiler_params=_TC2_PARAMS,
    )(out_a, r1, aggt_b, w2at, b2a2, w2bt, b2b2)
    return out


# speedup vs baseline: 1.2543x; 1.0332x over previous
import jax
import jax.numpy as jnp
from jax import lax
from jax.experimental import pallas as pl
from jax.experimental.pallas import tpu as pltpu
from jax.experimental.pallas import tpu_sc as plsc

N = 50000
D_FEAT = 128
DEG = 32
D_EDGE = 16
OUT_HALF = 64
MID = 96
BLK = 2048

_LANES = 128
_NPAD = ((N + _LANES - 1) // _LANES) * _LANES
_CH = _NPAD // _LANES
_NW = 32
_TRIPS = (_CH + _NW - 1) // _NW
_HROWS = DEG * D_EDGE // 2


def _make_sc_agg(ch0, nch):
    trips = (nch + _NW - 1) // _NW

    def _sc_agg(x_hbm, agg_hbm, buf0, buf1, ob0, ob1,
                isem0, isem1, osem0, osem1):
        w = lax.axis_index("s") * 2 + lax.axis_index("c")
        bufs = (buf0, buf1)
        obs = (ob0, ob1)
        isems = (isem0, isem1)
        osems = (osem0, osem1)

        def chunk(t):
            return w + _NW * t

        def start_in(t, h, b):
            pltpu.async_copy(
                x_hbm.at[pl.ds(h * _HROWS, _HROWS),
                         pl.ds((ch0 + chunk(t)) * _LANES, _LANES)],
                bufs[b], isems[b])

        @pl.when(chunk(0) < nch)
        def _():
            start_in(0, 0, 0)

        for t in range(trips):
            ob = obs[t % 2]
            for h in range(2):
                b = (2 * t + h) % 2
                nt, nh = (t, 1) if h == 0 else (t + 1, 0)
                if nt < trips:
                    @pl.when(chunk(nt) < nch)
                    def _(nt=nt, nh=nh, nb=1 - b):
                        start_in(nt, nh, nb)

                @pl.when(chunk(t) < nch)
                def _(t=t, h=h, b=b, ob=ob):
                    pltpu.make_async_copy(
                        x_hbm.at[pl.ds(0, _HROWS), pl.ds(0, _LANES)],
                        bufs[b], isems[b]).wait()
                    if h == 0 and t >= 2:
                        pltpu.make_async_copy(
                            ob, agg_hbm.at[:, pl.ds(0, _LANES)],
                            osems[t % 2]).wait()

                    def gbody(g, carry):
                        off = g * D_EDGE

                        def fbody(f4, carry2):
                            for df in range(4):
                                f = f4 * 4 + df
                                acc4 = []
                                for k in range(4):
                                    a = bufs[b][k * D_EDGE + f, pl.ds(off, D_EDGE)]
                                    for m in range(1, 4):
                                        a = a + bufs[b][(m * 4 + k) * D_EDGE + f,
                                                        pl.ds(off, D_EDGE)]
                                    acc4.append(a)
                                s = (acc4[0] + acc4[1]) + (acc4[2] + acc4[3])
                                if h == 0:
                                    ob[f, pl.ds(off, D_EDGE)] = s
                                else:
                                    ob[f, pl.ds(off, D_EDGE)] = (
                                        ob[f, pl.ds(off, D_EDGE)] + s)
                            return carry2

                        lax.fori_loop(0, D_EDGE // 4, fbody, 0)
                        return carry

                    lax.fori_loop(0, _LANES // D_EDGE, gbody, 0)
                    if h == 1:
                        pltpu.async_copy(
                            ob, agg_hbm.at[:, pl.ds(chunk(t) * _LANES, _LANES)],
                            osems[t % 2])

        for b in range(2):
            pltpu.make_async_copy(
                obs[b], agg_hbm.at[:, pl.ds(0, _LANES)], osems[b]).wait()

    return _sc_agg


def _sc_aggregate(x, ch0, nch):
    mesh = plsc.VectorSubcoreMesh(core_axis_name="c", subcore_axis_name="s")
    return pl.kernel(
        _make_sc_agg(ch0, nch),
        out_type=jax.ShapeDtypeStruct((D_EDGE, nch * _LANES), jnp.float32),
        mesh=mesh,
        scratch_types=[
            pltpu.VMEM((_HROWS, _LANES), jnp.float32),
            pltpu.VMEM((_HROWS, _LANES), jnp.float32),
            pltpu.VMEM((D_EDGE, _LANES), jnp.float32),
            pltpu.VMEM((D_EDGE, _LANES), jnp.float32),
            pltpu.SemaphoreType.DMA,
            pltpu.SemaphoreType.DMA,
            pltpu.SemaphoreType.DMA,
            pltpu.SemaphoreType.DMA,
        ],
    )(x)


TC1_BLK = 4096
BLK_A = 14
BLK_B = 4
CH_A = BLK_A * BLK // _LANES
CH_B = BLK_B * BLK // _LANES
_OFF_C = BLK_A + BLK_B


def _full(shape):
    return pl.BlockSpec(shape, lambda i: (0,) * len(shape))


def _tc1_body(nf, w1at, b1a, w1bt, b1b, r1out):
    h = jnp.maximum(
        jnp.dot(nf[...], w1at[...], preferred_element_type=jnp.float32) + b1a[...], 0.0)
    r1out[...] = jnp.tanh(
        jnp.dot(h, w1bt[...], preferred_element_type=jnp.float32)
        + b1b[...]).astype(jnp.bfloat16)


def _finish(r1, aggt, w2at, b2a, w2bt, b2b, out):
    agg = aggt.T
    h2 = jnp.maximum(
        jnp.dot(agg, w2at[...], preferred_element_type=jnp.float32) + b2a[...], 0.0)
    r2 = jnp.tanh(jnp.dot(h2, w2bt[...], preferred_element_type=jnp.float32) + b2b[...])
    res = jnp.concatenate([r1[...].astype(jnp.float32), r2], axis=1)
    inv = jax.lax.rsqrt(jnp.sum(res * res, axis=1, keepdims=True))
    out[...] = res * inv


def _tc2c_body(prev, r1, xs, w2at, b2a, w2bt, b2b, out):
    del prev
    mb = xs[...]
    parts = [mb[j * D_EDGE:(j + 1) * D_EDGE, :] for j in range(DEG)]
    while len(parts) > 1:
        parts = [parts[i] + parts[i + 1] for i in range(0, len(parts), 2)]
    _finish(r1, parts[0], w2at, b2a, w2bt, b2b, out)


def _tc2_body(prev, r1, aggt, w2at, b2a, w2bt, b2b, out):
    del prev
    _finish(r1, aggt[...], w2at, b2a, w2bt, b2b, out)


_TC_PARAMS = pltpu.CompilerParams(dimension_semantics=("parallel",))
_TC2_PARAMS = pltpu.CompilerParams(
    dimension_semantics=("parallel",), fuse_transposed_lhs_in_matmul=True)


def kernel(node_features, mailbox, W1a, b1a, W1b, b1b, W2a, b2a, W2b, b2b):
    n = node_features.shape[0]

    x = mailbox.transpose(1, 2, 0).reshape(DEG * D_EDGE, n)
    aggt_a = _sc_aggregate(x, 0, CH_A)
    aggt_b = _sc_aggregate(x, CH_A, CH_B)

    w2at = W2a.T
    b2a2 = b2a.reshape(1, OUT_HALF)
    w2bt = W2b.T
    b2b2 = b2b.reshape(1, OUT_HALF)

    r1 = pl.pallas_call(
        _tc1_body,
        grid=(pl.cdiv(n, TC1_BLK),),
        in_specs=[
            pl.BlockSpec((TC1_BLK, D_FEAT), lambda i: (i, 0)),
            _full((D_FEAT, MID)),
            _full((1, MID)),
            _full((MID, OUT_HALF)),
            _full((1, OUT_HALF)),
        ],
        out_specs=pl.BlockSpec((TC1_BLK, OUT_HALF), lambda i: (i, 0)),
        out_shape=jax.ShapeDtypeStruct((n, OUT_HALF), jnp.bfloat16),
        compiler_params=_TC_PARAMS,
    )(node_features,
      W1a.T, b1a.reshape(1, MID),
      W1b.T, b1b.reshape(1, OUT_HALF))

    nc_blocks = pl.cdiv(n - _OFF_C * BLK, BLK)
    out_c = pl.pallas_call(
        _tc2c_body,
        grid=(nc_blocks,),
        in_specs=[
            pl.BlockSpec((8, D_FEAT), lambda i: (0, 0)),
            pl.BlockSpec((BLK, OUT_HALF), lambda i: (i + _OFF_C, 0)),
            pl.BlockSpec((DEG * D_EDGE, BLK), lambda i: (0, i + _OFF_C)),
            _full((D_EDGE, OUT_HALF)),
            _full((1, OUT_HALF)),
            _full((OUT_HALF, OUT_HALF)),
            _full((1, OUT_HALF)),
        ],
        out_specs=pl.BlockSpec((BLK, D_FEAT), lambda i: (i + _OFF_C, 0)),
        out_shape=jax.ShapeDtypeStruct((n, D_FEAT), jnp.float32),
        input_output_aliases={},
        compiler_params=_TC2_PARAMS,
    )(node_features, r1, x, w2at, b2a2, w2bt, b2b2)

    out_a = pl.pallas_call(
        _tc2_body,
        grid=(BLK_A,),
        in_specs=[
            pl.BlockSpec((8, D_FEAT), lambda i: (0, 0)),
            pl.BlockSpec((BLK, OUT_HALF), lambda i: (i, 0)),
            pl.BlockSpec((D_EDGE, BLK), lambda i: (0, i)),
            _full((D_EDGE, OUT_HALF)),
            _full((1, OUT_HALF)),
            _full((OUT_HALF, OUT_HALF)),
            _full((1, OUT_HALF)),
        ],
        out_specs=pl.BlockSpec((BLK, D_FEAT), lambda i: (i, 0)),
        out_shape=jax.ShapeDtypeStruct((n, D_FEAT), jnp.float32),
        input_output_aliases={0: 0},
        compiler_params=_TC2_PARAMS,
    )(out_c, r1, aggt_a, w2at, b2a2, w2bt, b2b2)

    out = pl.pallas_call(
        _tc2_body,
        grid=(BLK_B,),
        in_specs=[
            pl.BlockSpec((8, D_FEAT), lambda i: (0, 0)),
            pl.BlockSpec((BLK, OUT_HALF), lambda i: (i + BLK_A, 0)),
            pl.BlockSpec((D_EDGE, BLK), lambda i: (0, i)),
            _full((D_EDGE, OUT_HALF)),
            _full((1, OUT_HALF)),
            _full((OUT_HALF, OUT_HALF)),
            _full((1, OUT_HALF)),
        ],
        out_specs=pl.BlockSpec((BLK, D_FEAT), lambda i: (i + BLK_A, 0)),
        out_shape=jax.ShapeDtypeStruct((n, D_FEAT), jnp.float32),
        input_output_aliases={0: 0},
        compiler_params=_TC2_PARAMS,
    )(out_a, r1, aggt_b, w2at, b2a2, w2bt, b2b2)
    return out
